# Initial kernel scaffold; baseline (speedup 1.0000x reference)
import jax
import jax.numpy as jnp
from jax.experimental import pallas as pl


def kernel(actions, action_embed, learned_token):
    raise NotImplementedError("write your pallas kernel here")


# speedup vs baseline: 1.3742x; 1.3742x over previous
import functools

import jax
import jax.numpy as jnp
from jax import lax
from jax.experimental import pallas as pl
from jax.experimental.pallas import tpu as pltpu
from jax.experimental.pallas import tpu_sc as plsc

NUM_CORES = 2
NUM_SUBCORES = 16
LANES = 16
NUM_WORKERS = NUM_CORES * NUM_SUBCORES

EMBED_DIM = 64
NUM_TOKENS = 2
CHUNK = 128


def _make_kernel(num_idx: int):
    assert num_idx % (NUM_WORKERS * CHUNK) == 0
    per_worker = num_idx // NUM_WORKERS
    n_chunks = per_worker // CHUNK
    vregs_per_row = EMBED_DIM // LANES

    mesh = plsc.VectorSubcoreMesh(
        core_axis_name="c", subcore_axis_name="s",
        num_cores=NUM_CORES, num_subcores=NUM_SUBCORES)

    @functools.partial(
        pl.kernel,
        out_type=jax.ShapeDtypeStruct((num_idx * NUM_TOKENS, EMBED_DIM),
                                      jnp.float32),
        mesh=mesh,
        scratch_types=[
            pltpu.VMEM((CHUNK,), jnp.int32),
            pltpu.VMEM((CHUNK, EMBED_DIM), jnp.float32),
            pltpu.VMEM((CHUNK * NUM_TOKENS, EMBED_DIM), jnp.float32),
            pltpu.VMEM((NUM_TOKENS, EMBED_DIM), jnp.float32),
            pltpu.SemaphoreType.DMA,
        ],
        compiler_params=pltpu.CompilerParams(use_tc_tiling_on_sc=False),
    )
    def action_encode(actions_hbm, table_hbm, lt_hbm, out_hbm,
                      idx_v, rows_v, out_v, lt_v, sem):
        wid = lax.axis_index("s") * NUM_CORES + lax.axis_index("c")
        base = wid * per_worker

        pltpu.sync_copy(lt_hbm, lt_v)
        lt0 = [lt_v[0, pl.ds(q * LANES, LANES)] for q in range(vregs_per_row)]
        lt1 = [lt_v[1, pl.ds(q * LANES, LANES)] for q in range(vregs_per_row)]

        def chunk_body(c, carry):
            start = base + c * CHUNK
            pltpu.sync_copy(actions_hbm.at[pl.ds(start, CHUNK)], idx_v)
            pltpu.async_copy(table_hbm.at[idx_v], rows_v, sem).wait()

            def row_body(j, carry2):
                for q in range(vregs_per_row):
                    r = rows_v[j, pl.ds(q * LANES, LANES)]
                    out_v[2 * j, pl.ds(q * LANES, LANES)] = r + lt0[q]
                    out_v[2 * j + 1, pl.ds(q * LANES, LANES)] = r + lt1[q]
                return carry2

            lax.fori_loop(0, CHUNK, row_body, 0, unroll=2)
            pltpu.sync_copy(
                out_v, out_hbm.at[pl.ds(NUM_TOKENS * start, NUM_TOKENS * CHUNK)])
            return carry

        lax.fori_loop(0, n_chunks, chunk_body, 0)

    return action_encode


def kernel(actions, action_embed, learned_token):
    b, t = actions.shape
    num_idx = b * t
    flat_idx = actions.reshape(num_idx).astype(jnp.int32)
    lt = learned_token.reshape(NUM_TOKENS, EMBED_DIM)
    out = _make_kernel(num_idx)(flat_idx, action_embed, lt)
    return out.reshape(b, t, NUM_TOKENS, EMBED_DIM)


# speedup vs baseline: 1.5630x; 1.1373x over previous
import functools

import jax
import jax.numpy as jnp
from jax import lax
from jax.experimental import pallas as pl
from jax.experimental.pallas import tpu as pltpu
from jax.experimental.pallas import tpu_sc as plsc

NUM_CORES = 2
NUM_SUBCORES = 16
LANES = 16
NUM_WORKERS = NUM_CORES * NUM_SUBCORES

EMBED_DIM = 64
NUM_TOKENS = 2
CHUNK = 128
NBUF = 2


def _make_kernel(num_idx: int):
    assert num_idx % (NUM_WORKERS * CHUNK) == 0
    per_worker = num_idx // NUM_WORKERS
    n_chunks = per_worker // CHUNK
    assert n_chunks % NBUF == 0 and n_chunks >= 3 * NBUF
    vregs_per_row = EMBED_DIM // LANES

    mesh = plsc.VectorSubcoreMesh(
        core_axis_name="c", subcore_axis_name="s",
        num_cores=NUM_CORES, num_subcores=NUM_SUBCORES)

    @functools.partial(
        pl.kernel,
        out_type=jax.ShapeDtypeStruct((num_idx * NUM_TOKENS, EMBED_DIM),
                                      jnp.float32),
        mesh=mesh,
        scratch_types=[
            pltpu.VMEM((n_chunks, CHUNK), jnp.int32),
            pltpu.VMEM((NBUF, CHUNK, EMBED_DIM), jnp.float32),
            pltpu.VMEM((NBUF, CHUNK * NUM_TOKENS, EMBED_DIM), jnp.float32),
            pltpu.VMEM((NUM_TOKENS, EMBED_DIM), jnp.float32),
            pltpu.SemaphoreType.DMA,
            pltpu.SemaphoreType.DMA,
            pltpu.SemaphoreType.DMA,
            pltpu.SemaphoreType.DMA,
        ],
        compiler_params=pltpu.CompilerParams(use_tc_tiling_on_sc=False),
    )
    def action_encode(actions_hbm, table_hbm, lt_hbm, out_hbm,
                      idx_v, rows_v, out_v, lt_v,
                      sem_g0, sem_g1, sem_w0, sem_w1):
        wid = lax.axis_index("s") * NUM_CORES + lax.axis_index("c")
        chunk0 = wid * n_chunks
        out_base = wid * per_worker * NUM_TOKENS
        sem_g = [sem_g0, sem_g1]
        sem_w = [sem_w0, sem_w1]

        pltpu.sync_copy(lt_hbm, lt_v)
        pltpu.sync_copy(actions_hbm.at[pl.ds(chunk0, n_chunks)], idx_v)
        lt0 = [lt_v[0, pl.ds(q * LANES, LANES)] for q in range(vregs_per_row)]
        lt1 = [lt_v[1, pl.ds(q * LANES, LANES)] for q in range(vregs_per_row)]

        def issue_gather(c, b):
            pltpu.async_copy(table_hbm.at[idx_v.at[c]], rows_v.at[b], sem_g[b])

        def wait_gather(c, b):
            pltpu.make_async_copy(
                table_hbm.at[idx_v.at[c]], rows_v.at[b], sem_g[b]).wait()

        def out_slice(c):
            return out_hbm.at[
                pl.ds(out_base + c * NUM_TOKENS * CHUNK, NUM_TOKENS * CHUNK)]

        def issue_wb(c, b):
            pltpu.async_copy(out_v.at[b], out_slice(c), sem_w[b])

        def wait_wb(c, b):
            pltpu.make_async_copy(out_v.at[b], out_slice(c), sem_w[b]).wait()

        def compute(b):
            rows_b = rows_v.at[b]
            out_b = out_v.at[b]

            def row_body(j, carry):
                for q in range(vregs_per_row):
                    r = rows_b[j, pl.ds(q * LANES, LANES)]
                    out_b[2 * j, pl.ds(q * LANES, LANES)] = r + lt0[q]
                    out_b[2 * j + 1, pl.ds(q * LANES, LANES)] = r + lt1[q]
                return carry

            lax.fori_loop(0, CHUNK, row_body, 0, unroll=4)

        def step(c, b, do_wait_wb, do_gather_ahead):
            wait_gather(c, b)
            if do_wait_wb:
                wait_wb(c - NBUF, b)
            compute(b)
            issue_wb(c, b)
            if do_gather_ahead:
                issue_gather(c + NBUF, b)

        for b in range(NBUF):
            issue_gather(b, b)
        for b in range(NBUF):
            step(b, b, do_wait_wb=False, do_gather_ahead=True)

        @pl.loop(NBUF, n_chunks - NBUF, step=NBUF)
        def _steady(c0):
            for b in range(NBUF):
                step(c0 + b, b, do_wait_wb=True, do_gather_ahead=True)

        for b in range(NBUF):
            step(n_chunks - NBUF + b, b, do_wait_wb=True, do_gather_ahead=False)
        for b in range(NBUF):
            wait_wb(n_chunks - NBUF + b, b)

    return action_encode


def kernel(actions, action_embed, learned_token):
    b, t = actions.shape
    num_idx = b * t
    flat_idx = actions.reshape(num_idx // CHUNK, CHUNK).astype(jnp.int32)
    lt = learned_token.reshape(NUM_TOKENS, EMBED_DIM)
    out = _make_kernel(num_idx)(flat_idx, action_embed, lt)
    return out.reshape(b, t, NUM_TOKENS, EMBED_DIM)


# speedup vs baseline: 1.6116x; 1.0311x over previous
import functools

import jax
import jax.numpy as jnp
from jax import lax
from jax.experimental import pallas as pl
from jax.experimental.pallas import tpu as pltpu
from jax.experimental.pallas import tpu_sc as plsc

NUM_CORES = 2
NUM_SUBCORES = 16
LANES = 16
NUM_WORKERS = NUM_CORES * NUM_SUBCORES

VOCAB = 100000
EMBED_DIM = 64
NUM_TOKENS = 2
B, T = 4096, 50

VA = (VOCAB // 128) * 128
NBLK = VA // 128
TAIL_ROWS = (VOCAB - VA) // 2
SCR_ROWS = VOCAB // 2
BLK_PER_SUB = -(-NBLK // NUM_SUBCORES)
BPW = B // NUM_WORKERS


def _make_kernel():
    mesh = plsc.VectorSubcoreMesh(
        core_axis_name="c", subcore_axis_name="s",
        num_cores=NUM_CORES, num_subcores=NUM_SUBCORES)

    @functools.partial(
        pl.kernel,
        out_type=(
            jax.ShapeDtypeStruct((T, NUM_TOKENS, EMBED_DIM, B), jnp.float32),
            jax.ShapeDtypeStruct((NUM_CORES, SCR_ROWS, 128), jnp.float32),
        ),
        mesh=mesh,
        scratch_types=[
            pltpu.VMEM((EMBED_DIM, 128), jnp.float32),
            pltpu.VMEM((EMBED_DIM, 128), jnp.float32),
            pltpu.VMEM((TAIL_ROWS, 128), jnp.float32),
            pltpu.VMEM((T, BPW), jnp.int32),
            pltpu.VMEM((T, BPW), jnp.int32),
            pltpu.VMEM((T, BPW), jnp.int32),
            pltpu.VMEM((2, BPW, 128), jnp.float32),
            pltpu.VMEM((2, NUM_TOKENS, EMBED_DIM, BPW), jnp.float32),
            pltpu.VMEM((NUM_TOKENS, EMBED_DIM), jnp.float32),
            pltpu.SemaphoreType.DMA,
            pltpu.SemaphoreType.DMA,
            pltpu.SemaphoreType.DMA,
            pltpu.SemaphoreType.DMA,
        ],
        compiler_params=pltpu.CompilerParams(needs_layout_passes=False),
    )
    def action_encode(actions_hbm, table_hbm, lt_hbm, tail_hbm,
                      out_hbm, scr_hbm,
                      slab_v, pair_v, tail_v, idx_v, ihalf_v, offv_v,
                      rows_v, out_v, lt_v,
                      sem_g0, sem_g1, sem_w0, sem_w1):
        cid = lax.axis_index("c")
        sid = lax.axis_index("s")
        sem_g = [sem_g0, sem_g1]
        sem_w = [sem_w0, sem_w1]

        iota = lax.iota(jnp.int32, LANES)
        rvec = [iota + (q * LANES) for q in range(8)]

        my_scr = scr_hbm.at[cid]

        @pl.loop(0, BLK_PER_SUB)
        def _blocks(jj):
            j = jj * NUM_SUBCORES + sid

            @pl.when(j < NBLK)
            def _do():
                pltpu.sync_copy(table_hbm.at[:, pl.ds(j * 128, 128)], slab_v)

                def r_body(r, carry):
                    c0 = jnp.full((LANES,), 2 * r, jnp.int32)
                    c1 = c0 + 1
                    for q in range(8):
                        cvec = c0 if q < 4 else c1
                        g = plsc.load_gather(slab_v, [rvec[q % 4], cvec])
                        pair_v[r, pl.ds(q * LANES, LANES)] = g
                    return carry

                lax.fori_loop(0, EMBED_DIM, r_body, 0)
                pltpu.sync_copy(pair_v, my_scr.at[pl.ds(j * EMBED_DIM,
                                                        EMBED_DIM)])

            return None

        @pl.when(sid == 0)
        def _tail():
            pltpu.sync_copy(tail_hbm, tail_v)
            pltpu.sync_copy(tail_v, my_scr.at[pl.ds(VA // 2, TAIL_ROWS)])

        plsc.subcore_barrier()

        w = sid * NUM_CORES + cid
        b0 = w * BPW

        pltpu.sync_copy(lt_hbm, lt_v)
        pltpu.sync_copy(actions_hbm.at[:, pl.ds(b0, BPW)], idx_v)

        def prep_body(t, carry):
            for q in range(BPW // LANES):
                v = idx_v[t, pl.ds(q * LANES, LANES)]
                ihalf_v[t, pl.ds(q * LANES, LANES)] = v >> 1
                offv_v[t, pl.ds(q * LANES, LANES)] = (v & 1) << 6
            return carry

        lax.fori_loop(0, T, prep_body, 0)

        zvec = jnp.zeros((LANES,), jnp.int32)
        onevec = jnp.full((LANES,), 1, jnp.int32)

        def issue_gather(t, b):
            pltpu.async_copy(my_scr.at[ihalf_v.at[t]], rows_v.at[b], sem_g[b])

        def wait_gather(t, b):
            pltpu.make_async_copy(
                my_scr.at[ihalf_v.at[t]], rows_v.at[b], sem_g[b]).wait()

        def out_slice(t):
            return out_hbm.at[t, :, :, pl.ds(b0, BPW)]

        def issue_wb(t, b):
            pltpu.async_copy(out_v.at[b], out_slice(t), sem_w[b])

        def wait_wb(t, b):
            pltpu.make_async_copy(out_v.at[b], out_slice(t), sem_w[b]).wait()

        def compute(t, b):
            rows_b = rows_v.at[b]
            out_b = out_v.at[b]
            offs = [offv_v[t, pl.ds(bb * LANES, LANES)]
                    for bb in range(BPW // LANES)]

            def e_body(e, carry):
                es = jnp.full((LANES,), e, jnp.int32)
                lt0 = plsc.load_gather(lt_v, [zvec, es])
                lt1 = plsc.load_gather(lt_v, [onevec, es])
                for bb in range(BPW // LANES):
                    col = offs[bb] + es
                    g = plsc.load_gather(rows_b, [rvec[bb % 4] + ((bb // 4) * 4 * LANES), col])
                    out_b[0, e, pl.ds(bb * LANES, LANES)] = g + lt0
                    out_b[1, e, pl.ds(bb * LANES, LANES)] = g + lt1
                return carry

            lax.fori_loop(0, EMBED_DIM, e_body, 0)

        def step(t, b, do_wait_wb, do_gather_ahead):
            if do_gather_ahead:
                issue_gather(t + 1, 1 - b)
            wait_gather(t, b)
            if do_wait_wb:
                wait_wb(t - 2, b)
            compute(t, b)
            issue_wb(t, b)

        issue_gather(0, 0)
        step(0, 0, do_wait_wb=False, do_gather_ahead=True)
        step(1, 1, do_wait_wb=False, do_gather_ahead=True)

        @pl.loop(2, T - 2, step=2)
        def _steady(t0):
            step(t0, 0, do_wait_wb=True, do_gather_ahead=True)
            step(t0 + 1, 1, do_wait_wb=True, do_gather_ahead=True)

        step(T - 2, 0, do_wait_wb=True, do_gather_ahead=True)
        step(T - 1, 1, do_wait_wb=True, do_gather_ahead=False)
        wait_wb(T - 2, 0)
        wait_wb(T - 1, 1)

    return action_encode


def kernel(actions, action_embed, learned_token):
    actions_t = actions.T.astype(jnp.int32)
    table_t = action_embed.T
    lt = learned_token.reshape(NUM_TOKENS, EMBED_DIM)
    tail_pair = action_embed[VA:].reshape(TAIL_ROWS, 128)
    out, _ = _make_kernel()(actions_t, table_t, lt, tail_pair)
    return jnp.transpose(out, (3, 0, 1, 2))
